# baseline (speedup 1.0000x reference)
import jax
import jax.numpy as jnp
from jax.experimental import pallas as pl
from jax.experimental.pallas import tpu as pltpu

_GB = 4096
_CH = 8


def _loss_block(o_ref, t_ref, acc_ref):
    c_idx = jax.lax.broadcasted_iota(jnp.int32, (1, 3, 1), 1)
    w_ang = jnp.where(c_idx == 0, 0.0, 1.0)
    w3 = jnp.where(c_idx == 0, 0.0, 3.0)

    acc = jnp.zeros((3, 128), jnp.float32)
    for c in range(_GB // _CH):
        o = o_ref[c * _CH:(c + 1) * _CH, :, :]
        t = t_ref[c * _CH:(c + 1) * _CH, :, :]
        fl = jnp.floor(o)
        adiff = (o - w_ang * fl) - t
        r = jnp.rint(adiff)
        d = adiff - w_ang * r
        contrib = d * d + w3 * jnp.abs(fl)
        acc = acc + jnp.sum(contrib, axis=0)

    acc_ref[...] = acc.reshape(1, 3, 128)


def kernel(outputs, targets):
    B = outputs.shape[0]
    o3 = outputs.reshape(-1, 128, 3).transpose(0, 2, 1)
    t3 = targets.reshape(-1, 128, 3).transpose(0, 2, 1)
    n = o3.shape[0]
    grid = n // _GB

    acc_p = pl.pallas_call(
        _loss_block,
        grid=(grid,),
        in_specs=[
            pl.BlockSpec((_GB, 3, 128), lambda i: (i, 0, 0)),
            pl.BlockSpec((_GB, 3, 128), lambda i: (i, 0, 0)),
        ],
        out_specs=pl.BlockSpec((1, 3, 128), lambda i: (i, 0, 0)),
        out_shape=jax.ShapeDtypeStruct((grid, 3, 128), jnp.float32),
        compiler_params=pltpu.CompilerParams(
            dimension_semantics=("arbitrary",),
        ),
    )(o3, t3)

    return jnp.sum(acc_p) / (B * 3)


# speedup vs baseline: 1.0146x; 1.0146x over previous
import jax
import jax.numpy as jnp
from jax.experimental import pallas as pl
from jax.experimental.pallas import tpu as pltpu

_GB = 2048
_CH = 8


def _loss_block(o_ref, t_ref, acc_ref):
    c_idx = jax.lax.broadcasted_iota(jnp.int32, (1, 3, 1), 1)
    w_ang = jnp.where(c_idx == 0, 0.0, 1.0)
    w3 = jnp.where(c_idx == 0, 0.0, 3.0)

    acc = jnp.zeros((3, 128), jnp.float32)
    for c in range(_GB // _CH):
        o = o_ref[c * _CH:(c + 1) * _CH, :, :]
        t = t_ref[c * _CH:(c + 1) * _CH, :, :]
        fl = jnp.floor(o)
        adiff = (o - w_ang * fl) - t
        r = jnp.rint(adiff)
        d = adiff - w_ang * r
        contrib = d * d + w3 * jnp.abs(fl)
        acc = acc + jnp.sum(contrib, axis=0)

    acc_ref[...] = acc.reshape(1, 3, 128)


def kernel(outputs, targets):
    B = outputs.shape[0]
    o3 = outputs.reshape(-1, 128, 3).transpose(0, 2, 1)
    t3 = targets.reshape(-1, 128, 3).transpose(0, 2, 1)
    n = o3.shape[0]
    grid = n // _GB

    acc_p = pl.pallas_call(
        _loss_block,
        grid=(grid,),
        in_specs=[
            pl.BlockSpec((_GB, 3, 128), lambda i: (i, 0, 0)),
            pl.BlockSpec((_GB, 3, 128), lambda i: (i, 0, 0)),
        ],
        out_specs=pl.BlockSpec((1, 3, 128), lambda i: (i, 0, 0)),
        out_shape=jax.ShapeDtypeStruct((grid, 3, 128), jnp.float32),
        compiler_params=pltpu.CompilerParams(
            dimension_semantics=("arbitrary",),
        ),
    )(o3, t3)

    return jnp.sum(acc_p) / (B * 3)
